# Initial kernel scaffold; baseline (speedup 1.0000x reference)
import jax
import jax.numpy as jnp
from jax.experimental import pallas as pl


def kernel(x, w_day, w_week, w_month):
    raise NotImplementedError("write your pallas kernel here")


# speedup vs baseline: 1.5871x; 1.5871x over previous
import jax
import jax.numpy as jnp
from jax import lax
from jax.experimental import pallas as pl
from jax.experimental.pallas import tpu as pltpu
from jax.experimental.pallas import tpu_sc as plsc

D = 128
LANES = 16
NC, NS = 2, 16
NW = NC * NS
C = 256
CJ = C // 128


def _body(xt_hbm, wd_hbm, ww_hbm, wm_hbm, out_hbm,
          idxd, idxw, idxm, rows_d, rows_w, rows_m, sem):
    n_tok = out_hbm.shape[0]
    per_w = n_tok // NW
    chunks = per_w // C
    wid = lax.axis_index("s") * NC + lax.axis_index("c")
    base0 = wid * per_w

    def chunk_body(g, _):
        base = base0 + g * C
        pltpu.sync_copy(xt_hbm.at[pl.ds(base, C)], idxd)
        pltpu.sync_copy(xt_hbm.at[pl.ds(n_tok + base, C)], idxw)
        pltpu.sync_copy(xt_hbm.at[pl.ds(2 * n_tok + base, C)], idxm)
        cps = []
        for j in range(CJ):
            src = pl.ds(j * 128, 128)
            dst = pl.ds(j * 128, 128)
            cps.append(pltpu.async_copy(wd_hbm.at[idxd.at[src]], rows_d.at[dst], sem))
            cps.append(pltpu.async_copy(ww_hbm.at[idxw.at[src]], rows_w.at[dst], sem))
            cps.append(pltpu.async_copy(wm_hbm.at[idxm.at[src]], rows_m.at[dst], sem))
        for cp in cps:
            cp.wait()

        def acc_body(t, _):
            for f in range(D // LANES):
                s = pl.ds(f * LANES, LANES)
                rows_d[t, s] = rows_d[t, s] + rows_w[t, s] + rows_m[t, s]
            return ()

        lax.fori_loop(0, C, acc_body, (), unroll=False)
        pltpu.sync_copy(rows_d, out_hbm.at[pl.ds(base, C)])
        return ()

    lax.fori_loop(0, chunks, chunk_body, (), unroll=False)


def kernel(x, w_day, w_week, w_month):
    b, l, _ = x.shape
    n_tok = b * l
    xt = x.transpose(2, 0, 1).reshape(3 * n_tok)
    mesh = plsc.VectorSubcoreMesh(core_axis_name="c", subcore_axis_name="s",
                                  num_cores=NC, num_subcores=NS)
    run = pl.kernel(
        _body,
        out_type=jax.ShapeDtypeStruct((n_tok, D), jnp.float32),
        mesh=mesh,
        scratch_types=[
            pltpu.VMEM((C,), jnp.int32),
            pltpu.VMEM((C,), jnp.int32),
            pltpu.VMEM((C,), jnp.int32),
            pltpu.VMEM((C, D), jnp.float32),
            pltpu.VMEM((C, D), jnp.float32),
            pltpu.VMEM((C, D), jnp.float32),
            pltpu.SemaphoreType.DMA,
        ],
    )
    out = run(xt, w_day, w_week, w_month)
    return out.reshape(b, l, D)


# speedup vs baseline: 10.8917x; 6.8626x over previous
import jax
import jax.numpy as jnp
from jax import lax
from jax.experimental import pallas as pl
from jax.experimental.pallas import tpu as pltpu
from jax.experimental.pallas import tpu_sc as plsc

D = 128
LANES = 16
NC, NS = 2, 16
NW = NC * NS
C = 256
CJ = C // 128
NV = 7
NT = NV * NV * NV


def _body(xt_hbm, wd_hbm, ww_hbm, wm_hbm, out_hbm, t_hbm,
          wd_v, ww_v, wm_v, t_v, idxd, idxw, idxm, idxc, rows, sem):
    n_tok = out_hbm.shape[0]
    per_w = n_tok // NW
    chunks = per_w // C
    wid = lax.axis_index("s") * NC + lax.axis_index("c")
    base0 = wid * per_w

    pltpu.sync_copy(wd_hbm.at[pl.ds(0, NV)], wd_v)
    pltpu.sync_copy(ww_hbm.at[pl.ds(0, NV)], ww_v)
    pltpu.sync_copy(wm_hbm.at[pl.ds(0, NV)], wm_v)

    def build_dw(dw, _):
        d = dw // NV
        w = dw - d * NV
        for m in range(NV):
            r = dw * NV + m
            for f in range(D // LANES):
                s = pl.ds(f * LANES, LANES)
                t_v[r, s] = wd_v[d, s] + ww_v[w, s] + wm_v[m, s]
        return ()

    lax.fori_loop(0, NV * NV, build_dw, (), unroll=False)
    pltpu.sync_copy(t_v, t_hbm)

    def chunk_body(g, _):
        base = base0 + g * C
        pltpu.sync_copy(xt_hbm.at[pl.ds(base, C)], idxd)
        pltpu.sync_copy(xt_hbm.at[pl.ds(n_tok + base, C)], idxw)
        pltpu.sync_copy(xt_hbm.at[pl.ds(2 * n_tok + base, C)], idxm)
        for i in range(C // LANES):
            s = pl.ds(i * LANES, LANES)
            idxc[s] = (idxd[s] * NV + idxw[s]) * NV + idxm[s]
        cps = []
        for j in range(CJ):
            s = pl.ds(j * 128, 128)
            cps.append(pltpu.async_copy(t_hbm.at[idxc.at[s]], rows.at[s], sem))
        for cp in cps:
            cp.wait()
        pltpu.sync_copy(rows, out_hbm.at[pl.ds(base, C)])
        return ()

    lax.fori_loop(0, chunks, chunk_body, (), unroll=False)


def kernel(x, w_day, w_week, w_month):
    b, l, _ = x.shape
    n_tok = b * l
    xt = x.transpose(2, 0, 1).reshape(3 * n_tok)
    mesh = plsc.VectorSubcoreMesh(core_axis_name="c", subcore_axis_name="s",
                                  num_cores=NC, num_subcores=NS)
    run = pl.kernel(
        _body,
        out_type=(jax.ShapeDtypeStruct((n_tok, D), jnp.float32),
                  jax.ShapeDtypeStruct((NT, D), jnp.float32)),
        mesh=mesh,
        scratch_types=[
            pltpu.VMEM((NV, D), jnp.float32),
            pltpu.VMEM((NV, D), jnp.float32),
            pltpu.VMEM((NV, D), jnp.float32),
            pltpu.VMEM((NT, D), jnp.float32),
            pltpu.VMEM((C,), jnp.int32),
            pltpu.VMEM((C,), jnp.int32),
            pltpu.VMEM((C,), jnp.int32),
            pltpu.VMEM((C,), jnp.int32),
            pltpu.VMEM((C, D), jnp.float32),
            pltpu.SemaphoreType.DMA,
        ],
    )
    out, _ = run(xt, w_day, w_week, w_month)
    return out.reshape(b, l, D)


# speedup vs baseline: 10.9780x; 1.0079x over previous
import jax
import jax.numpy as jnp
from jax import lax
from jax.experimental import pallas as pl
from jax.experimental.pallas import tpu as pltpu
from jax.experimental.pallas import tpu_sc as plsc

D = 128
LANES = 16
NC, NS = 2, 16
NW = NC * NS
C = 256
CJ = C // 128
NV = 7
NT = NV * NV * NV


def _body(xt_hbm, wd_hbm, ww_hbm, wm_hbm, out_hbm, t_hbm,
          wd_v, ww_v, wm_v, t_v, idxd, idxw, idxm, idxc,
          rows0, rows1, gsem, wsem0, wsem1):
    n_tok = out_hbm.shape[0]
    per_w = n_tok // NW
    chunks = per_w // C
    wid = lax.axis_index("s") * NC + lax.axis_index("c")
    base0 = wid * per_w
    rows = (rows0, rows1)
    wsem = (wsem0, wsem1)

    pltpu.sync_copy(wd_hbm.at[pl.ds(0, NV)], wd_v)
    pltpu.sync_copy(ww_hbm.at[pl.ds(0, NV)], ww_v)
    pltpu.sync_copy(wm_hbm.at[pl.ds(0, NV)], wm_v)

    def build_dw(dw, _):
        d = dw // NV
        w = dw - d * NV
        for m in range(NV):
            r = dw * NV + m
            for f in range(D // LANES):
                s = pl.ds(f * LANES, LANES)
                t_v[r, s] = wd_v[d, s] + ww_v[w, s] + wm_v[m, s]
        return ()

    lax.fori_loop(0, NV * NV, build_dw, (), unroll=False)
    pltpu.sync_copy(t_v, t_hbm)

    def chunk_body(g2, _):
        for p in range(2):
            base = base0 + (g2 * 2 + p) * C

            @pl.when(g2 > 0)
            def _():
                pltpu.make_async_copy(
                    rows[p], out_hbm.at[pl.ds(base0, C)], wsem[p]).wait()

            pltpu.sync_copy(xt_hbm.at[pl.ds(base, C)], idxd)
            pltpu.sync_copy(xt_hbm.at[pl.ds(n_tok + base, C)], idxw)
            pltpu.sync_copy(xt_hbm.at[pl.ds(2 * n_tok + base, C)], idxm)
            for i in range(C // LANES):
                s = pl.ds(i * LANES, LANES)
                idxc[s] = (idxd[s] * NV + idxw[s]) * NV + idxm[s]
            cps = []
            for j in range(CJ):
                s = pl.ds(j * 128, 128)
                cps.append(pltpu.async_copy(
                    t_hbm.at[idxc.at[s]], rows[p].at[s], gsem))
            for cp in cps:
                cp.wait()
            pltpu.async_copy(rows[p], out_hbm.at[pl.ds(base, C)], wsem[p])
        return ()

    lax.fori_loop(0, chunks // 2, chunk_body, (), unroll=False)
    for p in range(2):
        pltpu.make_async_copy(rows[p], out_hbm.at[pl.ds(base0, C)], wsem[p]).wait()


def kernel(x, w_day, w_week, w_month):
    b, l, _ = x.shape
    n_tok = b * l
    xt = x.transpose(2, 0, 1).reshape(3 * n_tok)
    mesh = plsc.VectorSubcoreMesh(core_axis_name="c", subcore_axis_name="s",
                                  num_cores=NC, num_subcores=NS)
    run = pl.kernel(
        _body,
        out_type=(jax.ShapeDtypeStruct((n_tok, D), jnp.float32),
                  jax.ShapeDtypeStruct((NT, D), jnp.float32)),
        mesh=mesh,
        scratch_types=[
            pltpu.VMEM((NV, D), jnp.float32),
            pltpu.VMEM((NV, D), jnp.float32),
            pltpu.VMEM((NV, D), jnp.float32),
            pltpu.VMEM((NT, D), jnp.float32),
            pltpu.VMEM((C,), jnp.int32),
            pltpu.VMEM((C,), jnp.int32),
            pltpu.VMEM((C,), jnp.int32),
            pltpu.VMEM((C,), jnp.int32),
            pltpu.VMEM((C, D), jnp.float32),
            pltpu.VMEM((C, D), jnp.float32),
            pltpu.SemaphoreType.DMA,
            pltpu.SemaphoreType.DMA,
            pltpu.SemaphoreType.DMA,
        ],
    )
    out, _ = run(xt, w_day, w_week, w_month)
    return out.reshape(b, l, D)
